# baseline (speedup 1.0000x reference)
import functools

import jax
import jax.numpy as jnp
from jax import lax
from jax.experimental import pallas as pl
from jax.experimental.pallas import tpu as pltpu
from jax.experimental.pallas import tpu_sc as plsc

_B, _N, _D = 8, 50000, 128
_K = 200
_NC, _NS = 2, 16
_WPB = 4
_CHUNK = 64
_LAST = _K - _CHUNK

_mesh = plsc.VectorSubcoreMesh(core_axis_name="c", subcore_axis_name="s")


@functools.partial(
    pl.kernel,
    mesh=_mesh,
    out_type=(
        jax.ShapeDtypeStruct((_B * _K, _D), jnp.float32),
        jax.ShapeDtypeStruct((_B * _K, _D), jnp.float32),
    ),
    scratch_types=[
        pltpu.VMEM((_B, _K), jnp.int32),
        pltpu.VMEM((_B * _K,), jnp.int32),
        pltpu.VMEM_SHARED((_B * _K,), jnp.int32),
        pltpu.VMEM((_CHUNK,), jnp.int32),
        pltpu.VMEM((_CHUNK, _D), jnp.float32),
        pltpu.SemaphoreType.DMA,
    ],
)
def _sc_gather(
    table_hbm, idx_hbm, out_a, out_b,
    idx_stage, idx_flat, idx_shared, idx_v, rows_v, sem,
):
    sid = lax.axis_index("s")
    wid = sid * _NC + lax.axis_index("c")
    b = wid // _WPB
    w4 = wid % _WPB

    @pl.when(sid == 0)
    def _publish_indices():
        pltpu.sync_copy(idx_hbm, idx_stage)
        starts = list(range(0, _K - 15, 16)) + [_K - 16]
        for bb in range(_B):
            for st in starts:
                idx_flat[pl.ds(bb * _K + st, 16)] = idx_stage[bb, pl.ds(st, 16)]
        pltpu.sync_copy(idx_flat, idx_shared)

    plsc.subcore_barrier()

    start = jnp.minimum(w4 * _CHUNK, _LAST)
    pltpu.sync_copy(idx_shared.at[pl.ds(b * _K + start, _CHUNK)], idx_v)
    row_off = b * _N
    for j in range(_CHUNK // 16):
        sl = pl.ds(j * 16, 16)
        idx_v[sl] = idx_v[sl] + row_off
    pltpu.async_copy(table_hbm.at[idx_v], rows_v, sem).wait()
    base = b * _K + start
    cp_a = pltpu.async_copy(rows_v, out_a.at[pl.ds(base, _CHUNK)], sem)
    cp_b = pltpu.async_copy(rows_v, out_b.at[pl.ds(base, _CHUNK)], sem)
    cp_a.wait()
    cp_b.wait()


def kernel(feature, tail_id):
    table = feature.reshape(_B * _N, _D)
    out_a, out_b = _sc_gather(table, tail_id)
    shape = (_B, _K, _D)
    return (out_a.reshape(shape), out_b.reshape(shape))


# speedup vs baseline: 1.0293x; 1.0293x over previous
import functools

import jax
import jax.numpy as jnp
from jax import lax
from jax.experimental import pallas as pl
from jax.experimental.pallas import tpu as pltpu
from jax.experimental.pallas import tpu_sc as plsc

_B, _N, _D = 8, 50000, 128
_K = 200
_NC, _NS = 2, 16
_WPB = 4
_CHUNK = 64
_HALF = _CHUNK // 2
_LAST = _K - _CHUNK

_mesh = plsc.VectorSubcoreMesh(core_axis_name="c", subcore_axis_name="s")


@functools.partial(
    pl.kernel,
    mesh=_mesh,
    out_type=(
        jax.ShapeDtypeStruct((_B * _K, _D), jnp.float32),
        jax.ShapeDtypeStruct((_B * _K, _D), jnp.float32),
    ),
    scratch_types=[
        pltpu.VMEM((_CHUNK,), jnp.int32),
        pltpu.VMEM((_CHUNK, _D), jnp.float32),
        pltpu.SemaphoreType.DMA,
        pltpu.SemaphoreType.DMA,
        pltpu.SemaphoreType.DMA,
    ],
)
def _sc_gather(table_hbm, idx_hbm, out_a, out_b, idx_v, rows_v, g0, g1, w):
    wid = lax.axis_index("s") * _NC + lax.axis_index("c")
    b = wid // _WPB
    w4 = wid % _WPB
    base = b * _K + jnp.minimum(w4 * _CHUNK, _LAST)
    pltpu.sync_copy(idx_hbm.at[pl.ds(base, _CHUNK)], idx_v)
    row_off = b * _N
    for j in range(_CHUNK // 16):
        sl = pl.ds(j * 16, 16)
        idx_v[sl] = idx_v[sl] + row_off

    cp0 = pltpu.async_copy(
        table_hbm.at[idx_v.at[pl.ds(0, _HALF)]], rows_v.at[pl.ds(0, _HALF)], g0
    )
    cp1 = pltpu.async_copy(
        table_hbm.at[idx_v.at[pl.ds(_HALF, _HALF)]],
        rows_v.at[pl.ds(_HALF, _HALF)],
        g1,
    )
    cp0.wait()
    src0 = rows_v.at[pl.ds(0, _HALF)]
    pltpu.async_copy(src0, out_a.at[pl.ds(base, _HALF)], w)
    pltpu.async_copy(src0, out_b.at[pl.ds(base, _HALF)], w)
    cp1.wait()
    src1 = rows_v.at[pl.ds(_HALF, _HALF)]
    wa = pltpu.async_copy(src1, out_a.at[pl.ds(base + _HALF, _HALF)], w)
    wb = pltpu.async_copy(src1, out_b.at[pl.ds(base + _HALF, _HALF)], w)
    wa.wait()
    wb.wait()
    wa.wait()
    wb.wait()


def kernel(feature, tail_id):
    table = feature.reshape(_B * _N, _D)
    out_a, out_b = _sc_gather(table, tail_id.reshape(_B * _K))
    shape = (_B, _K, _D)
    return (out_a.reshape(shape), out_b.reshape(shape))


# speedup vs baseline: 1.0374x; 1.0079x over previous
import functools

import jax
import jax.numpy as jnp
from jax import lax
from jax.experimental import pallas as pl
from jax.experimental.pallas import tpu as pltpu
from jax.experimental.pallas import tpu_sc as plsc

_B, _N, _D = 8, 50000, 128
_K = 200
_NC, _NS = 2, 16
_WPB = 4
_CHUNK = 64
_LAST = _K - _CHUNK

_mesh = plsc.VectorSubcoreMesh(core_axis_name="c", subcore_axis_name="s")


@functools.partial(
    pl.kernel,
    mesh=_mesh,
    out_type=(
        jax.ShapeDtypeStruct((_B * _K, _D), jnp.float32),
        jax.ShapeDtypeStruct((_B * _K, _D), jnp.float32),
    ),
    scratch_types=[
        pltpu.VMEM((_CHUNK,), jnp.int32),
        pltpu.VMEM((_CHUNK, _D), jnp.float32),
        pltpu.SemaphoreType.DMA,
    ],
)
def _sc_gather(table_hbm, idx_hbm, out_a, out_b, idx_v, rows_v, sem):
    wid = lax.axis_index("s") * _NC + lax.axis_index("c")
    b = wid // _WPB
    w4 = wid % _WPB
    base = b * _K + jnp.minimum(w4 * _CHUNK, _LAST)
    pltpu.sync_copy(idx_hbm.at[pl.ds(base, _CHUNK)], idx_v)
    pltpu.async_copy(table_hbm.at[idx_v], rows_v, sem).wait()
    cp_a = pltpu.async_copy(rows_v, out_a.at[pl.ds(base, _CHUNK)], sem)
    cp_b = pltpu.async_copy(rows_v, out_b.at[pl.ds(base, _CHUNK)], sem)
    cp_a.wait()
    cp_b.wait()


def kernel(feature, tail_id):
    table = feature.reshape(_B * _N, _D)
    gidx = (tail_id + jnp.arange(_B, dtype=jnp.int32)[:, None] * _N).reshape(
        _B * _K
    )
    out_a, out_b = _sc_gather(table, gidx)
    shape = (_B, _K, _D)
    return (out_a.reshape(shape), out_b.reshape(shape))
